# Initial kernel scaffold; baseline (speedup 1.0000x reference)
import jax
import jax.numpy as jnp
from jax.experimental import pallas as pl


def kernel(observations, W1, b1, W2, b2, dnd_keys, dnd_values):
    raise NotImplementedError("write your pallas kernel here")


# speedup vs baseline: 31.9885x; 31.9885x over previous
import functools

import jax
import jax.numpy as jnp
from jax.experimental import pallas as pl
from jax.experimental.pallas import tpu as pltpu

A = 4
K = 100000
D_OBS = 512
D_HID = 512
D_KEY = 128
P = 50
DELTA = 1e-3
B = 1024

KP = 100352
KB = 2048
NK = KP // KB
BT = 128
NB = B // BT
PAD_KEY = 1e4
MAX_ITERS = 40
BM = 256


def _mlp_body(obs_ref, w1_ref, b1_ref, w2_ref, b2_ref, out_ref):
    h = jnp.dot(obs_ref[...], w1_ref[...], preferred_element_type=jnp.float32)
    h = jnp.maximum(h + b1_ref[...], 0.0)
    out_ref[...] = jnp.dot(h, w2_ref[...], preferred_element_type=jnp.float32) + b2_ref[...]


def _dnd_body(keys_ref, mk_ref, v_ref, out_ref, d2_ref):
    a = pl.program_id(0)
    bt = pl.program_id(1)
    kb = pl.program_id(2)
    q = keys_ref[...]
    mk = mk_ref[0]
    prod = jax.lax.dot_general(
        q, mk, (((1,), (1,)), ((), ())), preferred_element_type=jnp.float32
    )
    q2 = jnp.sum(q * q, axis=1, keepdims=True)
    m2 = jnp.sum(mk * mk, axis=1)
    d2_ref[:, pl.ds(kb * KB, KB)] = q2 - 2.0 * prod + m2[None, :]

    @pl.when(kb == NK - 1)
    def _select_and_reduce():
        BIG = 3.0e38

        def rep(x):
            return jax.lax.broadcast_in_dim(x, (BT, 128), (0, 1)) + jnp.zeros(
                (BT, 128), jnp.float32)

        def chunk(i):
            return d2_ref[:, pl.ds(i * KB, KB)]

        def mm_step(i, c):
            mn, mx = c
            blk = chunk(i)
            col = jax.lax.broadcasted_iota(jnp.int32, (BT, KB), 1) + i * KB
            mn = jnp.minimum(mn, rep(jnp.min(blk, axis=1, keepdims=True)))
            mx = jnp.maximum(mx, rep(jnp.max(
                jnp.where(col < K, blk, -BIG), axis=1, keepdims=True)))
            return mn, mx

        lo0, hi0 = jax.lax.fori_loop(
            0, NK, mm_step,
            (jnp.full((BT, 128), BIG), jnp.full((BT, 128), -BIG)))
        lo0 = lo0 - 1.0
        done0 = jnp.zeros((BT, 128), dtype=jnp.float32)

        def cond(state):
            it, _, _, done = state
            return jnp.logical_and(it < MAX_ITERS, jnp.min(done) < 0.5)

        def body(state):
            it, lo, hi, done = state
            mid = 0.5 * (lo + hi)

            def cstep(i, acc):
                blk = chunk(i)
                return acc + rep(jnp.sum(
                    jnp.where(blk <= mid[:, 0:1], 1.0, 0.0),
                    axis=1, keepdims=True))

            cnt = jax.lax.fori_loop(0, NK, cstep, jnp.zeros((BT, 128)))
            ge = cnt >= float(P)
            pend = done < 0.5
            new_hi = jnp.where(jnp.logical_and(ge, pend), mid, hi)
            new_lo = jnp.where(jnp.logical_and(jnp.logical_not(ge), pend), mid, lo)
            new_done = jnp.maximum(done, jnp.where(cnt == float(P), 1.0, 0.0))
            return it + 1, new_lo, new_hi, new_done

        _, _, t, _ = jax.lax.while_loop(cond, body, (0, lo0, hi0, done0))

        def fstep(i, c):
            nm, dn = c
            blk = chunk(i)
            w = jnp.where(blk <= t[:, 0:1], 1.0 / (blk + DELTA), 0.0)
            v = v_ref[0, :, pl.ds(i * KB, KB)]
            nm = nm + rep(jnp.sum(w * v, axis=1, keepdims=True))
            dn = dn + rep(jnp.sum(w, axis=1, keepdims=True))
            return nm, dn

        num, den = jax.lax.fori_loop(
            0, NK, fstep, (jnp.zeros((BT, 128)), jnp.zeros((BT, 128))))
        out_ref[...] = (num[:, 0:1] / den[:, 0:1]).reshape(1, 1, BT, 1)


@jax.jit
def kernel(observations, W1, b1, W2, b2, dnd_keys, dnd_values):
    keys = pl.pallas_call(
        _mlp_body,
        grid=(B // BM,),
        in_specs=[
            pl.BlockSpec((BM, D_OBS), lambda i: (i, 0)),
            pl.BlockSpec((D_OBS, D_HID), lambda i: (0, 0)),
            pl.BlockSpec((1, D_HID), lambda i: (0, 0)),
            pl.BlockSpec((D_HID, D_KEY), lambda i: (0, 0)),
            pl.BlockSpec((1, D_KEY), lambda i: (0, 0)),
        ],
        out_specs=pl.BlockSpec((BM, D_KEY), lambda i: (i, 0)),
        out_shape=jax.ShapeDtypeStruct((B, D_KEY), jnp.float32),
    )(observations, W1, b1[None, :], W2, b2[None, :])

    mk_pad = jnp.pad(dnd_keys, ((0, 0), (0, KP - K), (0, 0)),
                     constant_values=PAD_KEY)
    v_pad = jnp.pad(dnd_values, ((0, 0), (0, KP - K)))

    out = pl.pallas_call(
        _dnd_body,
        grid=(A, NB, NK),
        in_specs=[
            pl.BlockSpec((BT, D_KEY), lambda a, bt, kb: (bt, 0)),
            pl.BlockSpec((1, KB, D_KEY), lambda a, bt, kb: (a, kb, 0)),
            pl.BlockSpec((1, 1, KP), lambda a, bt, kb: (a, 0, 0)),
        ],
        out_specs=pl.BlockSpec((1, 1, BT, 1), lambda a, bt, kb: (a, bt, 0, 0)),
        out_shape=jax.ShapeDtypeStruct((A, NB, BT, 1), jnp.float32),
        scratch_shapes=[pltpu.VMEM((BT, KP), jnp.float32)],
    )(keys, mk_pad, v_pad[:, None, :])

    return out.reshape(A, B).T
